# baseline (speedup 1.0000x reference)
import jax
import jax.numpy as jnp
from jax import lax
from jax.experimental import pallas as pl
from jax.experimental.pallas import tpu as pltpu
from jax.experimental.pallas import tpu_sc as plsc

_N = 10000
_E = 320000
_D = 128

_NC = 2
_NS = 16
_NW = _NC * _NS

_CHUNK = 128
_CPW = 80
_EPW = _CPW * _CHUNK
_EPAD = _NW * _EPW
_NCH = _EPAD // _CHUNK
_NPAD = 10240
_RPT = _NPAD // _NS


def _make_seg_sum(with_counts):
    def body(feat_hbm, el_hbm, zeros_hbm, *rest):
        if with_counts:
            (z1_hbm, out_hbm, cnt_hbm, acc, idx0, idx1, rows0, rows1, cnt_v,
             g0, g1) = rest
        else:
            out_hbm, acc, idx0, idx1, rows0, rows1, g0, g1 = rest
        idx = (idx0, idx1)
        rows = (rows0, rows1)
        gsem = (g0, g1)
        c = lax.axis_index("c")
        s = lax.axis_index("s")
        wid = s * _NC + c
        r0 = s * _RPT
        pltpu.sync_copy(zeros_hbm.at[pl.ds(r0, _RPT)], acc.at[pl.ds(r0, _RPT)])
        if with_counts:
            pltpu.sync_copy(z1_hbm, cnt_v)
        plsc.subcore_barrier()
        ch0 = wid * _CPW

        def gather_start(b, ch):
            pltpu.sync_copy(el_hbm.at[ch], idx[b])
            pltpu.async_copy(feat_hbm.at[idx[b].at[0]], rows[b], gsem[b])

        def consume(b):
            pltpu.make_async_copy(
                feat_hbm.at[idx[b].at[0]], rows[b], gsem[b]).wait()
            if with_counts:
                ones = jnp.ones((16,), jnp.float32)
                for j in range(_CHUNK // 16):
                    plsc.addupdate_scatter(
                        cnt_v, [idx[b][1, pl.ds(j * 16, 16)]], ones)

        gather_start(0, ch0)
        gather_start(1, ch0 + 1)

        @pl.loop(0, _CPW - 2, step=2)
        def _steady(t0):
            for b in (0, 1):
                consume(b)
                gather_start(b, ch0 + t0 + b + 2)

        for b in (0, 1):
            consume(b)

        plsc.subcore_barrier()
        pltpu.sync_copy(acc.at[pl.ds(r0, _RPT)], out_hbm.at[c, pl.ds(r0, _RPT)])
        if with_counts:
            pltpu.sync_copy(cnt_v, cnt_hbm.at[c, s])

    out_type = [jax.ShapeDtypeStruct((_NC, _NPAD, _D), jnp.float32)]
    scratch = [
        pltpu.VMEM_SHARED((_NPAD, _D), jnp.float32),
        pltpu.VMEM((2, _CHUNK), jnp.int32),
        pltpu.VMEM((2, _CHUNK), jnp.int32),
        pltpu.VMEM((_CHUNK, _D), jnp.float32),
        pltpu.VMEM((_CHUNK, _D), jnp.float32),
    ]
    if with_counts:
        out_type.append(jax.ShapeDtypeStruct((_NC, _NS, _NPAD), jnp.float32))
        scratch.append(pltpu.VMEM((_NPAD,), jnp.float32))
    scratch.append(pltpu.SemaphoreType.DMA)
    scratch.append(pltpu.SemaphoreType.DMA)

    mesh = plsc.VectorSubcoreMesh(core_axis_name="c", subcore_axis_name="s")
    return pl.kernel(
        body, out_type=out_type, mesh=mesh, scratch_types=scratch,
        compiler_params=pltpu.CompilerParams(needs_layout_passes=False),
    )


_seg_cnt = _make_seg_sum(True)
_seg = _make_seg_sum(False)


def _layer1_body(acc_ref, cnt_ref, W_ref, b_ref, g_ref, bt_ref,
                 h_ref, dinv_ref):
    s = acc_ref[0, :_N, :] + acc_ref[1, :_N, :]
    cnt = jnp.sum(cnt_ref[:_N, :], axis=1, keepdims=True)
    dinv = 1.0 / jnp.maximum(cnt, 1.0)
    agg = s * dinv
    h = jnp.dot(agg, W_ref[...], preferred_element_type=jnp.float32) + b_ref[...]
    h = jnp.maximum(h, 0.0)
    m = jnp.mean(h, axis=-1, keepdims=True)
    d = h - m
    v = jnp.mean(d * d, axis=-1, keepdims=True)
    h_ref[...] = d * lax.rsqrt(v + 1e-5) * g_ref[...] + bt_ref[...]
    dinv_ref[...] = jnp.broadcast_to(dinv, (_N, _D))


def _layer2_body(acc_ref, dinv_ref, W_ref, b_ref, g_ref, bt_ref,
                 Wo_ref, bo_ref, out_ref):
    s = acc_ref[0, :_N, :] + acc_ref[1, :_N, :]
    agg = s * dinv_ref[...]
    h = jnp.dot(agg, W_ref[...], preferred_element_type=jnp.float32) + b_ref[...]
    h = jnp.maximum(h, 0.0)
    m = jnp.mean(h, axis=-1, keepdims=True)
    d = h - m
    v = jnp.mean(d * d, axis=-1, keepdims=True)
    h = d * lax.rsqrt(v + 1e-5) * g_ref[...] + bt_ref[...]
    out_ref[...] = (
        jnp.dot(h, Wo_ref[...], preferred_element_type=jnp.float32) + bo_ref[...]
    )


_tc_layer1 = pl.pallas_call(
    _layer1_body,
    out_shape=[
        jax.ShapeDtypeStruct((_N, _D), jnp.float32),
        jax.ShapeDtypeStruct((_N, _D), jnp.float32),
    ],
)

_tc_layer2 = pl.pallas_call(
    _layer2_body,
    out_shape=jax.ShapeDtypeStruct((_N, _D), jnp.float32),
)


@jax.jit
def kernel(x, edge_index, batch, W1, b1, g1, bt1, W2, b2, g2, bt2, Wo, bo):
    del batch
    pad = _EPAD - _E
    srcp = jnp.concatenate([edge_index[0], jnp.zeros((pad,), jnp.int32)])
    dstp = jnp.concatenate([edge_index[1], jnp.full((pad,), _N, jnp.int32)])
    el = jnp.stack([srcp.reshape(_NCH, _CHUNK), dstp.reshape(_NCH, _CHUNK)],
                   axis=1)
    z128 = jnp.zeros((_NPAD, _D), jnp.float32)
    z1 = jnp.zeros((_NPAD,), jnp.float32)

    acc1, cnt = _seg_cnt(x, el, z128, z1)
    cnt_t = cnt.reshape(_NW, _NPAD).T
    h1, dinv = _tc_layer1(acc1, cnt_t, W1, b1.reshape(1, _D),
                          g1.reshape(1, _D), bt1.reshape(1, _D))
    (acc2,) = _seg(h1, el, z128)
    return _tc_layer2(acc2, dinv, W2, b2.reshape(1, _D), g2.reshape(1, _D),
                      bt2.reshape(1, _D), Wo, bo.reshape(1, _D))


# speedup vs baseline: 3.4607x; 3.4607x over previous
import jax
import jax.numpy as jnp
from jax import lax
from jax.experimental import pallas as pl
from jax.experimental.pallas import tpu as pltpu
from jax.experimental.pallas import tpu_sc as plsc

_N = 10000
_E = 320000
_D = 128

_NC = 2
_NS = 16
_NW = _NC * _NS

_CHUNK = 128
_NCH = _E // _CHUNK
_CPW = _NCH // _NW
_XTRA = _NCH - _CPW * _NW
_NPAD = 10240
_RPT = _NPAD // _NS


def _make_seg_sum(with_counts):
    def body(feat_hbm, el_hbm, zeros_hbm, *rest):
        if with_counts:
            (z1_hbm, out_hbm, cnt_hbm, acc, idx0, idx1, rows0, rows1, cnt_v,
             g0, g1) = rest
        else:
            out_hbm, acc, idx0, idx1, rows0, rows1, g0, g1 = rest
        idx = (idx0, idx1)
        rows = (rows0, rows1)
        gsem = (g0, g1)
        c = lax.axis_index("c")
        s = lax.axis_index("s")
        wid = s * _NC + c
        r0 = s * _RPT
        pltpu.sync_copy(zeros_hbm.at[pl.ds(r0, _RPT)], acc.at[pl.ds(r0, _RPT)])
        if with_counts:
            pltpu.sync_copy(z1_hbm, cnt_v)
        plsc.subcore_barrier()
        ch0 = wid * _CPW

        def gather_start(b, ch):
            pltpu.sync_copy(el_hbm.at[ch], idx[b])
            pltpu.async_copy(feat_hbm.at[idx[b].at[0]], rows[b], gsem[b])

        def consume(b):
            pltpu.make_async_copy(
                feat_hbm.at[idx[b].at[0]], rows[b], gsem[b]).wait()
            if with_counts:
                ones = jnp.ones((16,), jnp.float32)
                for j in range(_CHUNK // 16):
                    plsc.addupdate_scatter(
                        cnt_v, [idx[b][1, pl.ds(j * 16, 16)]], ones)
            pltpu.sync_copy(rows[b], acc.at[idx[b].at[1]], add=True)

        gather_start(0, ch0)
        gather_start(1, ch0 + 1)

        @pl.loop(0, _CPW - 2, step=2)
        def _steady(t0):
            for b in (0, 1):
                consume(b)
                gather_start(b, ch0 + t0 + b + 2)

        for b in (0, 1):
            consume(b)

        @pl.when(wid < _XTRA)
        def _extra():
            gather_start(0, _CPW * _NW + wid)
            consume(0)

        plsc.subcore_barrier()
        pltpu.sync_copy(acc.at[pl.ds(r0, _RPT)], out_hbm.at[c, pl.ds(r0, _RPT)])
        if with_counts:
            pltpu.sync_copy(cnt_v, cnt_hbm.at[c, s])

    out_type = [jax.ShapeDtypeStruct((_NC, _NPAD, _D), jnp.float32)]
    scratch = [
        pltpu.VMEM_SHARED((_NPAD, _D), jnp.float32),
        pltpu.VMEM((2, _CHUNK), jnp.int32),
        pltpu.VMEM((2, _CHUNK), jnp.int32),
        pltpu.VMEM((_CHUNK, _D), jnp.float32),
        pltpu.VMEM((_CHUNK, _D), jnp.float32),
    ]
    if with_counts:
        out_type.append(jax.ShapeDtypeStruct((_NC, _NS, _NPAD), jnp.float32))
        scratch.append(pltpu.VMEM((_NPAD,), jnp.float32))
    scratch.append(pltpu.SemaphoreType.DMA)
    scratch.append(pltpu.SemaphoreType.DMA)

    mesh = plsc.VectorSubcoreMesh(core_axis_name="c", subcore_axis_name="s")
    return pl.kernel(
        body, out_type=out_type, mesh=mesh, scratch_types=scratch,
        compiler_params=pltpu.CompilerParams(needs_layout_passes=False),
    )


_seg_cnt = _make_seg_sum(True)
_seg = _make_seg_sum(False)


def _layer1_body(acc_ref, cnt_ref, W_ref, b_ref, g_ref, bt_ref,
                 h_ref, dinv_ref):
    s = acc_ref[0, :_N, :] + acc_ref[1, :_N, :]
    cnt = jnp.sum(cnt_ref[:_N, :], axis=1, keepdims=True)
    dinv = 1.0 / jnp.maximum(cnt, 1.0)
    agg = s * dinv
    h = jnp.dot(agg, W_ref[...], preferred_element_type=jnp.float32) + b_ref[...]
    h = jnp.maximum(h, 0.0)
    m = jnp.mean(h, axis=-1, keepdims=True)
    d = h - m
    v = jnp.mean(d * d, axis=-1, keepdims=True)
    h_ref[...] = d * lax.rsqrt(v + 1e-5) * g_ref[...] + bt_ref[...]
    dinv_ref[...] = jnp.broadcast_to(dinv, (_N, _D))


def _layer2_body(acc_ref, dinv_ref, W_ref, b_ref, g_ref, bt_ref,
                 Wo_ref, bo_ref, out_ref):
    s = acc_ref[0, :_N, :] + acc_ref[1, :_N, :]
    agg = s * dinv_ref[...]
    h = jnp.dot(agg, W_ref[...], preferred_element_type=jnp.float32) + b_ref[...]
    h = jnp.maximum(h, 0.0)
    m = jnp.mean(h, axis=-1, keepdims=True)
    d = h - m
    v = jnp.mean(d * d, axis=-1, keepdims=True)
    h = d * lax.rsqrt(v + 1e-5) * g_ref[...] + bt_ref[...]
    out_ref[...] = (
        jnp.dot(h, Wo_ref[...], preferred_element_type=jnp.float32) + bo_ref[...]
    )


_tc_layer1 = pl.pallas_call(
    _layer1_body,
    out_shape=[
        jax.ShapeDtypeStruct((_N, _D), jnp.float32),
        jax.ShapeDtypeStruct((_N, _D), jnp.float32),
    ],
)

_tc_layer2 = pl.pallas_call(
    _layer2_body,
    out_shape=jax.ShapeDtypeStruct((_N, _D), jnp.float32),
)


@jax.jit
def kernel(x, edge_index, batch, W1, b1, g1, bt1, W2, b2, g2, bt2, Wo, bo):
    del batch
    el = jnp.swapaxes(edge_index.reshape(2, _NCH, _CHUNK), 0, 1)
    z128 = jnp.zeros((_NPAD, _D), jnp.float32)
    z1 = jnp.zeros((_NPAD,), jnp.float32)

    acc1, cnt = _seg_cnt(x, el, z128, z1)
    cnt_t = cnt.reshape(_NW, _NPAD).T
    h1, dinv = _tc_layer1(acc1, cnt_t, W1, b1.reshape(1, _D),
                          g1.reshape(1, _D), bt1.reshape(1, _D))
    (acc2,) = _seg(h1, el, z128)
    return _tc_layer2(acc2, dinv, W2, b2.reshape(1, _D), g2.reshape(1, _D),
                      bt2.reshape(1, _D), Wo, bo.reshape(1, _D))
